# Initial kernel scaffold; baseline (speedup 1.0000x reference)
import jax
import jax.numpy as jnp
from jax.experimental import pallas as pl


def kernel(x, edge_index_0, edge_index_1, edge_index_2, W0_l, b0, W0_r, g0, beta0, W1_l, b1, W1_r, g1, beta1, W2_l, b2, W2_r, g2, beta2, W_lin1, b_lin1, W_lin2, b_lin2):
    raise NotImplementedError("write your pallas kernel here")


# speedup vs baseline: 3.1260x; 3.1260x over previous
import jax
import jax.numpy as jnp
import numpy as np
from jax import lax
from jax.experimental import pallas as pl
from jax.experimental.pallas import tpu as pltpu
from jax.experimental.pallas import tpu_sc as plsc

NC = 2
NS = 16
LANES = 16
CHUNK = 128
NW = NC * NS
PW = 128
BN_EPS = 1e-5


def _round_up(x, m):
    return (x + m - 1) // m * m


def _make_seg_sum(n_entries, n_acc, cdup):
    cpt = n_entries // (NW * CHUNK)
    chc = CHUNK // cdup
    rpt = n_acc // NS
    nz = (rpt + CHUNK - 1) // CHUNK
    mesh = plsc.VectorSubcoreMesh(core_axis_name="c", subcore_axis_name="s")

    def body(h_hbm, src_hbm, dst_hbm, cdst_hbm, sums_hbm, cnts_hbm,
             accum, cntacc, src_cur, dst_cur, cdst_cur, rows_v, ones_v,
             zrow_v, sem):
        cid = lax.axis_index("c")
        sid = lax.axis_index("s")
        wid = cid * NS + sid

        zs = jnp.zeros((LANES,), jnp.float32)
        os_ = jnp.ones((LANES,), jnp.float32)

        def fill_z(i, _):
            for k in range(PW // LANES):
                zrow_v[i, pl.ds(k * LANES, LANES)] = zs
            return _

        def fill_o(i, _):
            for k in range(PW // LANES):
                ones_v[i, pl.ds(k * LANES, LANES)] = os_
            return _

        lax.fori_loop(0, CHUNK, fill_z, None)
        lax.fori_loop(0, chc, fill_o, None)

        for z in range(nz):
            r = min(CHUNK, rpt - z * CHUNK)
            r0 = sid * rpt + z * CHUNK
            pltpu.sync_copy(zrow_v.at[pl.ds(0, r)], accum.at[pl.ds(r0, r)])
            pltpu.sync_copy(zrow_v.at[pl.ds(0, r)], cntacc.at[pl.ds(r0, r)])
        plsc.subcore_barrier()

        def step(j, _):
            off = (wid * cpt + j) * CHUNK
            offc = (wid * cpt + j) * chc
            pltpu.sync_copy(src_hbm.at[pl.ds(off, CHUNK)], src_cur)
            pltpu.sync_copy(dst_hbm.at[pl.ds(off, CHUNK)], dst_cur)
            pltpu.sync_copy(cdst_hbm.at[pl.ds(offc, chc)], cdst_cur)
            pltpu.async_copy(h_hbm.at[src_cur], rows_v, sem).wait()
            pltpu.sync_copy(rows_v, accum.at[dst_cur], add=True)
            pltpu.sync_copy(ones_v, cntacc.at[cdst_cur], add=True)
            return _

        lax.fori_loop(0, cpt, step, None)
        plsc.subcore_barrier()

        for z in range(nz):
            r = min(CHUNK, rpt - z * CHUNK)
            r0 = sid * rpt + z * CHUNK
            pltpu.sync_copy(accum.at[pl.ds(r0, r)], zrow_v.at[pl.ds(0, r)])
            pltpu.sync_copy(zrow_v.at[pl.ds(0, r)], sums_hbm.at[cid, pl.ds(r0, r)])
            pltpu.sync_copy(cntacc.at[pl.ds(r0, r)], zrow_v.at[pl.ds(0, r)])
            pltpu.sync_copy(zrow_v.at[pl.ds(0, r)], cnts_hbm.at[cid, pl.ds(r0, r)])

    return pl.kernel(
        body,
        out_type=(
            jax.ShapeDtypeStruct((NC, n_acc, PW), jnp.float32),
            jax.ShapeDtypeStruct((NC, n_acc, PW), jnp.float32),
        ),
        mesh=mesh,
        scratch_types=[
            pltpu.VMEM_SHARED((n_acc, PW), jnp.float32),
            pltpu.VMEM_SHARED((n_acc, PW), jnp.float32),
            pltpu.VMEM((CHUNK,), jnp.int32),
            pltpu.VMEM((CHUNK,), jnp.int32),
            pltpu.VMEM((chc,), jnp.int32),
            pltpu.VMEM((CHUNK, PW), jnp.float32),
            pltpu.VMEM((chc, PW), jnp.float32),
            pltpu.VMEM((CHUNK, PW), jnp.float32),
            pltpu.SemaphoreType.DMA,
        ],
    )


_BN_SCALE = np.float32(np.sqrt(1.0 + BN_EPS))


def _dot(a, b):
    return jax.lax.dot(a, b, precision=jax.lax.Precision.HIGHEST,
                       preferred_element_type=jnp.float32)


def _mean_from_partials(s_ref, c_ref, n, dup, n_acc):
    s = s_ref[0] + s_ref[1]
    cnt = c_ref[0, :, 0:1] + c_ref[1, :, 0:1]
    if dup == 2:
        s = jnp.concatenate([s[0:n], s[n_acc:n_acc + n]], axis=1)
    else:
        s = s[0:n]
    return s / jnp.maximum(cnt[0:n], 1.0)


def _tc0_body(s_ref, c_ref, x_ref, wl_ref, wr_ref, b_ref, g_ref, be_ref,
              o_ref):
    mean = _mean_from_partials(s_ref, c_ref, 4000, 1, 4096)
    xd = x_ref[0:4000]
    h = _dot(mean, wl_ref[...]) + _dot(xd, wr_ref[...]) + b_ref[...]
    h = (h / _BN_SCALE) * g_ref[...] + be_ref[...]
    h = jnp.where(h > 0, h, 0.2 * h)
    o_ref[0:4000] = h[:, 0:PW]
    o_ref[4000:8000] = h[:, PW:2 * PW]


def _tc1_body(s_ref, c_ref, h_ref, wl_ref, wr_ref, b_ref, g_ref, be_ref,
              w2l_ref, z_ref, hd_ref):
    mean = _mean_from_partials(s_ref, c_ref, 1500, 2, 1536)
    xd = jnp.concatenate([h_ref[0:1500], h_ref[4000:5500]], axis=1)
    h = _dot(mean, wl_ref[...]) + _dot(xd, wr_ref[...]) + b_ref[...]
    h = (h / _BN_SCALE) * g_ref[...] + be_ref[...]
    h = jnp.where(h > 0, h, 0.2 * h)
    z_ref[...] = _dot(h, w2l_ref[...])
    hd_ref[...] = h[0:512]


def _tc2_body(s_ref, c_ref, hd_ref, wr_ref, b_ref, g_ref, be_ref,
              w1_ref, b1_ref, w2_ref, b2_ref, o_ref):
    mean_t = _mean_from_partials(s_ref, c_ref, 512, 1, 640)
    h = mean_t + _dot(hd_ref[...], wr_ref[...]) + b_ref[...]
    h = (h / _BN_SCALE) * g_ref[...] + be_ref[...]
    h = _dot(h, w1_ref[...]) + b1_ref[...]
    h = jnp.where(h > 0, h, 0.2 * h)
    o_ref[...] = _dot(h, w2_ref[...]) + b2_ref[...]


def _prep_edges(ei, n_dst, dup, src_n, dst_n, group):
    e = ei.shape[1]
    e_pad = _round_up(e, group)
    src = ei[0].astype(jnp.int32)
    dst = ei[1].astype(jnp.int32)
    if e_pad != e:
        src = jnp.concatenate([src, jnp.zeros((e_pad - e,), jnp.int32)])
        dst = jnp.concatenate([dst, jnp.full((e_pad - e,), n_dst, jnp.int32)])
    cdst = dst
    if dup:
        src = (src[:, None] + jnp.array([0, src_n], jnp.int32)[None, :]).reshape(-1)
        dst = (dst[:, None] + jnp.array([0, dst_n], jnp.int32)[None, :]).reshape(-1)
    return src, dst, cdst


_SEG0 = _make_seg_sum(131072, 4096, 1)
_SEG1 = _make_seg_sum(98304, 3072, 2)
_SEG2 = _make_seg_sum(16384, 640, 1)


def kernel(x, edge_index_0, edge_index_1, edge_index_2, W0_l, b0, W0_r, g0,
           beta0, W1_l, b1, W1_r, g1, beta1, W2_l, b2, W2_r, g2, beta2,
           W_lin1, b_lin1, W_lin2, b_lin2):
    f32 = jnp.float32

    src0, dst0, cdst0 = _prep_edges(edge_index_0, 4000, False, 0, 0, NW * CHUNK)
    sums0, cnts0 = _SEG0(x, src0, dst0, cdst0)
    h1p = pl.pallas_call(
        _tc0_body,
        out_shape=jax.ShapeDtypeStruct((8000, PW), f32),
    )(sums0, cnts0, x, W0_l, W0_r, b0, g0, beta0)

    src1, dst1, cdst1 = _prep_edges(edge_index_1, 1500, True, 4000, 1536,
                                    NW * CHUNK // 2)
    sums1, cnts1 = _SEG1(h1p, src1, dst1, cdst1)
    z2, hd = pl.pallas_call(
        _tc1_body,
        out_shape=(
            jax.ShapeDtypeStruct((1500, PW), f32),
            jax.ShapeDtypeStruct((512, 2 * PW), f32),
        ),
    )(sums1, cnts1, h1p, W1_l, W1_r, b1, g1, beta1, W2_l)

    src2, dst2, cdst2 = _prep_edges(edge_index_2, 512, False, 0, 0, NW * CHUNK)
    sums2, cnts2 = _SEG2(z2, src2, dst2, cdst2)
    out = pl.pallas_call(
        _tc2_body,
        out_shape=jax.ShapeDtypeStruct((512, 1), f32),
    )(sums2, cnts2, hd, W2_r, b2, g2, beta2, W_lin1, b_lin1, W_lin2, b_lin2)
    return out


# speedup vs baseline: 4.0963x; 1.3104x over previous
import jax
import jax.numpy as jnp
import numpy as np
from jax import lax
from jax.experimental import pallas as pl
from jax.experimental.pallas import tpu as pltpu
from jax.experimental.pallas import tpu_sc as plsc

NC = 2
NS = 16
LANES = 16
CHUNK = 128
NW = NC * NS
PW = 128
BN_EPS = 1e-5


def _round_up(x, m):
    return (x + m - 1) // m * m


def _make_seg_sum(n_entries, n_acc, cdup):
    cpt = n_entries // (NW * CHUNK)
    chc = CHUNK // cdup
    rpt = n_acc // NS
    nz = (rpt + CHUNK - 1) // CHUNK
    mesh = plsc.VectorSubcoreMesh(core_axis_name="c", subcore_axis_name="s")

    def body(h_hbm, src_hbm, dst_hbm, cdst_hbm, sums_hbm, cnts_hbm,
             accum, cntacc, src_v, dst_all, cdst_all, dst_cur, cdst_cur,
             rows_a, rows_b, ones_v, gsem_a, gsem_b):
        cid = lax.axis_index("c")
        sid = lax.axis_index("s")
        wid = cid * NS + sid

        zs = jnp.zeros((LANES,), jnp.float32)
        os_ = jnp.ones((LANES,), jnp.float32)

        def fill_z(i, _):
            for k in range(PW // LANES):
                rows_a[i, pl.ds(k * LANES, LANES)] = zs
            return _

        def fill_o(i, _):
            for k in range(PW // LANES):
                ones_v[i, pl.ds(k * LANES, LANES)] = os_
            return _

        lax.fori_loop(0, CHUNK, fill_z, None)
        lax.fori_loop(0, chc, fill_o, None)

        pltpu.sync_copy(src_hbm.at[pl.ds(wid * cpt * CHUNK, cpt * CHUNK)], src_v)
        pltpu.sync_copy(dst_hbm.at[pl.ds(wid * cpt * CHUNK, cpt * CHUNK)], dst_all)
        if cdup != 1:
            pltpu.sync_copy(cdst_hbm.at[pl.ds(wid * cpt * chc, cpt * chc)],
                            cdst_all)

        for z in range(nz):
            r = min(CHUNK, rpt - z * CHUNK)
            r0 = sid * rpt + z * CHUNK
            pltpu.sync_copy(rows_a.at[pl.ds(0, r)], accum.at[pl.ds(r0, r)])
            pltpu.sync_copy(rows_a.at[pl.ds(0, r)], cntacc.at[pl.ds(r0, r)])
        plsc.subcore_barrier()

        def gref(j):
            return h_hbm.at[src_v.at[pl.ds(j * CHUNK, CHUNK)]]

        def start_gather(j, buf, sem):
            pltpu.async_copy(gref(j), buf, sem)

        def wait_gather(j, buf, sem):
            pltpu.make_async_copy(gref(j), buf, sem).wait()

        def scatter_chunk(j, buf):
            csrc = dst_all if cdup == 1 else cdst_all
            for k in range(CHUNK // LANES):
                dst_cur[pl.ds(k * LANES, LANES)] = (
                    dst_all[pl.ds(j * CHUNK + k * LANES, LANES)])
            for k in range(chc // LANES):
                cdst_cur[pl.ds(k * LANES, LANES)] = (
                    csrc[pl.ds(j * chc + k * LANES, LANES)])
            pltpu.sync_copy(buf, accum.at[dst_cur], add=True)
            pltpu.sync_copy(ones_v, cntacc.at[cdst_cur], add=True)

        start_gather(0, rows_a, gsem_a)

        def pair(k, _):
            j0 = 2 * k
            start_gather(j0 + 1, rows_b, gsem_b)
            wait_gather(j0, rows_a, gsem_a)
            scatter_chunk(j0, rows_a)
            start_gather(j0 + 2, rows_a, gsem_a)
            wait_gather(j0 + 1, rows_b, gsem_b)
            scatter_chunk(j0 + 1, rows_b)
            return _

        lax.fori_loop(0, cpt // 2 - 1, pair, None)
        jt = cpt - 2
        start_gather(jt + 1, rows_b, gsem_b)
        wait_gather(jt, rows_a, gsem_a)
        scatter_chunk(jt, rows_a)
        wait_gather(jt + 1, rows_b, gsem_b)
        scatter_chunk(jt + 1, rows_b)
        plsc.subcore_barrier()

        for z in range(nz):
            r = min(CHUNK, rpt - z * CHUNK)
            r0 = sid * rpt + z * CHUNK
            pltpu.sync_copy(accum.at[pl.ds(r0, r)], rows_a.at[pl.ds(0, r)])
            pltpu.sync_copy(rows_a.at[pl.ds(0, r)], sums_hbm.at[cid, pl.ds(r0, r)])
            pltpu.sync_copy(cntacc.at[pl.ds(r0, r)], rows_a.at[pl.ds(0, r)])
            pltpu.sync_copy(rows_a.at[pl.ds(0, r)], cnts_hbm.at[cid, pl.ds(r0, r)])

    return pl.kernel(
        body,
        out_type=(
            jax.ShapeDtypeStruct((NC, n_acc, PW), jnp.float32),
            jax.ShapeDtypeStruct((NC, n_acc, PW), jnp.float32),
        ),
        mesh=mesh,
        scratch_types=[
            pltpu.VMEM_SHARED((n_acc, PW), jnp.float32),
            pltpu.VMEM_SHARED((n_acc, PW), jnp.float32),
            pltpu.VMEM((cpt * CHUNK,), jnp.int32),
            pltpu.VMEM((cpt * CHUNK,), jnp.int32),
            pltpu.VMEM((cpt * chc if cdup != 1 else LANES,), jnp.int32),
            pltpu.VMEM((CHUNK,), jnp.int32),
            pltpu.VMEM((chc,), jnp.int32),
            pltpu.VMEM((CHUNK, PW), jnp.float32),
            pltpu.VMEM((CHUNK, PW), jnp.float32),
            pltpu.VMEM((chc, PW), jnp.float32),
            pltpu.SemaphoreType.DMA,
            pltpu.SemaphoreType.DMA,
        ],
    )


_BN_SCALE = np.float32(np.sqrt(1.0 + BN_EPS))


def _dot(a, b):
    return jax.lax.dot(a, b, precision=jax.lax.Precision.HIGHEST,
                       preferred_element_type=jnp.float32)


def _mean_from_partials(s_ref, c_ref, n, dup, n_acc):
    s = s_ref[0] + s_ref[1]
    cnt = c_ref[0, :, 0:1] + c_ref[1, :, 0:1]
    if dup == 2:
        s = jnp.concatenate([s[0:n], s[n_acc:n_acc + n]], axis=1)
    else:
        s = s[0:n]
    return s / jnp.maximum(cnt[0:n], 1.0)


def _tc0_body(s_ref, c_ref, x_ref, wl_ref, wr_ref, b_ref, g_ref, be_ref,
              o_ref):
    mean = _mean_from_partials(s_ref, c_ref, 4000, 1, 4096)
    xd = x_ref[0:4000]
    h = _dot(mean, wl_ref[...]) + _dot(xd, wr_ref[...]) + b_ref[...]
    h = (h / _BN_SCALE) * g_ref[...] + be_ref[...]
    h = jnp.where(h > 0, h, 0.2 * h)
    o_ref[0:4000] = h[:, 0:PW]
    o_ref[4000:8000] = h[:, PW:2 * PW]


def _tc1_body(s_ref, c_ref, h_ref, wl_ref, wr_ref, b_ref, g_ref, be_ref,
              w2l_ref, z_ref, hd_ref):
    mean = _mean_from_partials(s_ref, c_ref, 1500, 2, 1536)
    xd = jnp.concatenate([h_ref[0:1500], h_ref[4000:5500]], axis=1)
    h = _dot(mean, wl_ref[...]) + _dot(xd, wr_ref[...]) + b_ref[...]
    h = (h / _BN_SCALE) * g_ref[...] + be_ref[...]
    h = jnp.where(h > 0, h, 0.2 * h)
    z_ref[...] = _dot(h, w2l_ref[...])
    hd_ref[...] = h[0:512]


def _tc2_body(s_ref, c_ref, hd_ref, wr_ref, b_ref, g_ref, be_ref,
              w1_ref, b1_ref, w2_ref, b2_ref, o_ref):
    mean_t = _mean_from_partials(s_ref, c_ref, 512, 1, 640)
    h = mean_t + _dot(hd_ref[...], wr_ref[...]) + b_ref[...]
    h = (h / _BN_SCALE) * g_ref[...] + be_ref[...]
    h = _dot(h, w1_ref[...]) + b1_ref[...]
    h = jnp.where(h > 0, h, 0.2 * h)
    o_ref[...] = _dot(h, w2_ref[...]) + b2_ref[...]


def _prep_edges(ei, n_dst, dup, src_n, dst_n, group, spread):
    e = ei.shape[1]
    e_pad = _round_up(e, group)
    src = ei[0].astype(jnp.int32)
    dst = ei[1].astype(jnp.int32)
    if e_pad != e:
        pad_dst = n_dst + jnp.arange(e_pad - e, dtype=jnp.int32) % spread
        src = jnp.concatenate([src, jnp.zeros((e_pad - e,), jnp.int32)])
        dst = jnp.concatenate([dst, pad_dst])
    cdst = dst
    if dup:
        src = (src[:, None] + jnp.array([0, src_n], jnp.int32)[None, :]).reshape(-1)
        dst = (dst[:, None] + jnp.array([0, dst_n], jnp.int32)[None, :]).reshape(-1)
    return src, dst, cdst


_SEG0 = _make_seg_sum(131072, 4096, 1)
_SEG1 = _make_seg_sum(98304, 3072, 2)
_SEG2 = _make_seg_sum(16384, 640, 1)


def kernel(x, edge_index_0, edge_index_1, edge_index_2, W0_l, b0, W0_r, g0,
           beta0, W1_l, b1, W1_r, g1, beta1, W2_l, b2, W2_r, g2, beta2,
           W_lin1, b_lin1, W_lin2, b_lin2):
    f32 = jnp.float32

    src0, dst0, cdst0 = _prep_edges(edge_index_0, 4000, False, 0, 0,
                                    NW * CHUNK, 64)
    sums0, cnts0 = _SEG0(x, src0, dst0, cdst0)
    h1p = pl.pallas_call(
        _tc0_body,
        out_shape=jax.ShapeDtypeStruct((8000, PW), f32),
    )(sums0, cnts0, x, W0_l, W0_r, b0, g0, beta0)

    src1, dst1, cdst1 = _prep_edges(edge_index_1, 1500, True, 4000, 1536,
                                    NW * CHUNK // 2, 32)
    sums1, cnts1 = _SEG1(h1p, src1, dst1, cdst1)
    z2, hd = pl.pallas_call(
        _tc1_body,
        out_shape=(
            jax.ShapeDtypeStruct((1500, PW), f32),
            jax.ShapeDtypeStruct((512, 2 * PW), f32),
        ),
    )(sums1, cnts1, h1p, W1_l, W1_r, b1, g1, beta1, W2_l)

    src2, dst2, cdst2 = _prep_edges(edge_index_2, 512, False, 0, 0,
                                    NW * CHUNK, 64)
    sums2, cnts2 = _SEG2(z2, src2, dst2, cdst2)
    out = pl.pallas_call(
        _tc2_body,
        out_shape=jax.ShapeDtypeStruct((512, 1), f32),
    )(sums2, cnts2, hd, W2_r, b2, g2, beta2, W_lin1, b_lin1, W_lin2, b_lin2)
    return out


# speedup vs baseline: 8.3935x; 2.0491x over previous
import jax
import jax.numpy as jnp
import numpy as np
from jax import lax
from jax.experimental import pallas as pl
from jax.experimental.pallas import tpu as pltpu
from jax.experimental.pallas import tpu_sc as plsc

NC = 2
NS = 16
LANES = 16
CHUNK = 128
NW = NC * NS
PW = 128
BN_EPS = 1e-5


def _round_up(x, m):
    return (x + m - 1) // m * m


def _make_seg_sum(n_entries, n_acc, cdup):
    cpt = n_entries // (NW * CHUNK)
    chc = CHUNK // cdup
    rpt = n_acc // NS
    nz = (rpt + CHUNK - 1) // CHUNK
    mesh = plsc.VectorSubcoreMesh(core_axis_name="c", subcore_axis_name="s")

    def body(h_hbm, src_hbm, dst_hbm, cdst_hbm, sums_hbm, cnts_hbm,
             accum, cntacc, src_v, dst_all, cdst_all, dst_cur, cdst_cur,
             rows_a, rows_b, ones_v, gsem_a, gsem_b):
        cid = lax.axis_index("c")
        sid = lax.axis_index("s")
        wid = cid * NS + sid

        zs = jnp.zeros((LANES,), jnp.float32)
        os_ = jnp.ones((LANES,), jnp.float32)

        def fill_z(i, _):
            for k in range(PW // LANES):
                rows_a[i, pl.ds(k * LANES, LANES)] = zs
            return _

        def fill_o(i, _):
            for k in range(PW // LANES):
                ones_v[i, pl.ds(k * LANES, LANES)] = os_
            return _

        lax.fori_loop(0, CHUNK, fill_z, None)
        lax.fori_loop(0, chc, fill_o, None)

        pltpu.sync_copy(src_hbm.at[pl.ds(wid * cpt * CHUNK, cpt * CHUNK)], src_v)
        pltpu.sync_copy(dst_hbm.at[pl.ds(wid * cpt * CHUNK, cpt * CHUNK)], dst_all)
        if cdup != 1:
            pltpu.sync_copy(cdst_hbm.at[pl.ds(wid * cpt * chc, cpt * chc)],
                            cdst_all)

        for z in range(nz):
            r = min(CHUNK, rpt - z * CHUNK)
            r0 = sid * rpt + z * CHUNK
            pltpu.sync_copy(rows_a.at[pl.ds(0, r)], accum.at[pl.ds(r0, r)])
            pltpu.sync_copy(rows_a.at[pl.ds(0, r)], cntacc.at[pl.ds(r0, r)])
        plsc.subcore_barrier()

        def gref(j):
            return h_hbm.at[src_v.at[pl.ds(j * CHUNK, CHUNK)]]

        def start_gather(j, buf, sem):
            pltpu.async_copy(gref(j), buf, sem)

        def wait_gather(j, buf, sem):
            pltpu.make_async_copy(gref(j), buf, sem).wait()

        def scatter_chunk(j, buf):
            csrc = dst_all if cdup == 1 else cdst_all
            for k in range(CHUNK // LANES):
                dst_cur[pl.ds(k * LANES, LANES)] = (
                    dst_all[pl.ds(j * CHUNK + k * LANES, LANES)])
            for k in range(chc // LANES):
                cdst_cur[pl.ds(k * LANES, LANES)] = (
                    csrc[pl.ds(j * chc + k * LANES, LANES)])
            pltpu.sync_copy(buf, accum.at[dst_cur], add=True)
            pltpu.sync_copy(ones_v, cntacc.at[cdst_cur], add=True)

        start_gather(0, rows_a, gsem_a)

        def pair(k, _):
            j0 = 2 * k
            start_gather(j0 + 1, rows_b, gsem_b)
            wait_gather(j0, rows_a, gsem_a)
            scatter_chunk(j0, rows_a)
            start_gather(j0 + 2, rows_a, gsem_a)
            wait_gather(j0 + 1, rows_b, gsem_b)
            scatter_chunk(j0 + 1, rows_b)
            return _

        lax.fori_loop(0, cpt // 2 - 1, pair, None)
        jt = cpt - 2
        start_gather(jt + 1, rows_b, gsem_b)
        wait_gather(jt, rows_a, gsem_a)
        scatter_chunk(jt, rows_a)
        wait_gather(jt + 1, rows_b, gsem_b)
        scatter_chunk(jt + 1, rows_b)
        plsc.subcore_barrier()

        for z in range(nz):
            r = min(CHUNK, rpt - z * CHUNK)
            r0 = sid * rpt + z * CHUNK
            pltpu.sync_copy(accum.at[pl.ds(r0, r)], rows_a.at[pl.ds(0, r)])
            pltpu.sync_copy(rows_a.at[pl.ds(0, r)], sums_hbm.at[cid, pl.ds(r0, r)])
            pltpu.sync_copy(cntacc.at[pl.ds(r0, r)], rows_a.at[pl.ds(0, r)])
            pltpu.sync_copy(rows_a.at[pl.ds(0, r)], cnts_hbm.at[cid, pl.ds(r0, r)])

    return pl.kernel(
        body,
        out_type=(
            jax.ShapeDtypeStruct((NC, n_acc, PW), jnp.float32),
            jax.ShapeDtypeStruct((NC, n_acc, PW), jnp.float32),
        ),
        mesh=mesh,
        scratch_types=[
            pltpu.VMEM_SHARED((n_acc, PW), jnp.float32),
            pltpu.VMEM_SHARED((n_acc, PW), jnp.float32),
            pltpu.VMEM((cpt * CHUNK,), jnp.int32),
            pltpu.VMEM((cpt * CHUNK,), jnp.int32),
            pltpu.VMEM((cpt * chc if cdup != 1 else LANES,), jnp.int32),
            pltpu.VMEM((CHUNK,), jnp.int32),
            pltpu.VMEM((chc,), jnp.int32),
            pltpu.VMEM((CHUNK, PW), jnp.float32),
            pltpu.VMEM((CHUNK, PW), jnp.float32),
            pltpu.VMEM((chc, PW), jnp.float32),
            pltpu.SemaphoreType.DMA,
            pltpu.SemaphoreType.DMA,
        ],
    )


_BN_SCALE = np.float32(np.sqrt(1.0 + BN_EPS))


def _dot(a, b):
    return jax.lax.dot(a, b, precision=jax.lax.Precision.HIGHEST,
                       preferred_element_type=jnp.float32)


def _mean_from_partials(s_ref, c_ref, n, dup, n_acc):
    s = s_ref[0] + s_ref[1]
    cnt = c_ref[0, :, 0:1] + c_ref[1, :, 0:1]
    if dup == 2:
        s = jnp.concatenate([s[0:n], s[n_acc:n_acc + n]], axis=1)
    else:
        s = s[0:n]
    return s / jnp.maximum(cnt[0:n], 1.0)


def _tc0_body(s_ref, c_ref, x_ref, wl_ref, wr_ref, b_ref, g_ref, be_ref,
              o_ref):
    mean = _mean_from_partials(s_ref, c_ref, 4000, 1, 4096)
    xd = x_ref[0:4000]
    h = _dot(mean, wl_ref[...]) + _dot(xd, wr_ref[...]) + b_ref[...]
    h = (h / _BN_SCALE) * g_ref[...] + be_ref[...]
    h = jnp.where(h > 0, h, 0.2 * h)
    o_ref[0:4000] = h[:, 0:PW]
    o_ref[4000:8000] = h[:, PW:2 * PW]


def _tc1_body(s_ref, c_ref, h_ref, wl_ref, wr_ref, b_ref, g_ref, be_ref,
              w2l_ref, z_ref, hd_ref):
    mean = _mean_from_partials(s_ref, c_ref, 1500, 2, 1536)
    xd = jnp.concatenate([h_ref[0:1500], h_ref[4000:5500]], axis=1)
    h = _dot(mean, wl_ref[...]) + _dot(xd, wr_ref[...]) + b_ref[...]
    h = (h / _BN_SCALE) * g_ref[...] + be_ref[...]
    h = jnp.where(h > 0, h, 0.2 * h)
    z_ref[...] = _dot(h, w2l_ref[...])
    hd_ref[...] = h[0:512]


def _tc2_body(s_ref, c_ref, hd_ref, wr_ref, b_ref, g_ref, be_ref,
              w1_ref, b1_ref, w2_ref, b2_ref, o_ref):
    mean_t = _mean_from_partials(s_ref, c_ref, 512, 1, 640)
    h = mean_t + _dot(hd_ref[...], wr_ref[...]) + b_ref[...]
    h = (h / _BN_SCALE) * g_ref[...] + be_ref[...]
    h = _dot(h, w1_ref[...]) + b1_ref[...]
    h = jnp.where(h > 0, h, 0.2 * h)
    o_ref[...] = _dot(h, w2_ref[...]) + b2_ref[...]


def _prep_edges(ei, n_dst, dup, src_n, dst_n, group, spread):
    e = ei.shape[1]
    e_pad = _round_up(e, group)
    src = ei[0].astype(jnp.int32)
    dst = ei[1].astype(jnp.int32)
    if e_pad != e:
        pad_iota = jnp.arange(e_pad - e, dtype=jnp.int32)
        src = jnp.concatenate([src, pad_iota % 1024])
        dst = jnp.concatenate([dst, n_dst + pad_iota % spread])
    cdst = dst
    if dup:
        src = (src[:, None] + jnp.array([0, src_n], jnp.int32)[None, :]).reshape(-1)
        dst = (dst[:, None] + jnp.array([0, dst_n], jnp.int32)[None, :]).reshape(-1)
    return src, dst, cdst


_SEG0 = _make_seg_sum(131072, 4096, 1)
_SEG1 = _make_seg_sum(98304, 3072, 2)
_SEG2 = _make_seg_sum(16384, 640, 1)


def kernel(x, edge_index_0, edge_index_1, edge_index_2, W0_l, b0, W0_r, g0,
           beta0, W1_l, b1, W1_r, g1, beta1, W2_l, b2, W2_r, g2, beta2,
           W_lin1, b_lin1, W_lin2, b_lin2):
    f32 = jnp.float32

    src0, dst0, cdst0 = _prep_edges(edge_index_0, 4000, False, 0, 0,
                                    NW * CHUNK, 64)
    sums0, cnts0 = _SEG0(x, src0, dst0, cdst0)
    h1p = pl.pallas_call(
        _tc0_body,
        out_shape=jax.ShapeDtypeStruct((8000, PW), f32),
    )(sums0, cnts0, x, W0_l, W0_r, b0, g0, beta0)

    src1, dst1, cdst1 = _prep_edges(edge_index_1, 1500, True, 4000, 1536,
                                    NW * CHUNK // 2, 32)
    sums1, cnts1 = _SEG1(h1p, src1, dst1, cdst1)
    z2, hd = pl.pallas_call(
        _tc1_body,
        out_shape=(
            jax.ShapeDtypeStruct((1500, PW), f32),
            jax.ShapeDtypeStruct((512, 2 * PW), f32),
        ),
    )(sums1, cnts1, h1p, W1_l, W1_r, b1, g1, beta1, W2_l)

    src2, dst2, cdst2 = _prep_edges(edge_index_2, 512, False, 0, 0,
                                    NW * CHUNK, 64)
    sums2, cnts2 = _SEG2(z2, src2, dst2, cdst2)
    out = pl.pallas_call(
        _tc2_body,
        out_shape=jax.ShapeDtypeStruct((512, 1), f32),
    )(sums2, cnts2, hd, W2_r, b2, g2, beta2, W_lin1, b_lin1, W_lin2, b_lin2)
    return out


# speedup vs baseline: 9.0424x; 1.0773x over previous
import jax
import jax.numpy as jnp
import numpy as np
from jax import lax
from jax.experimental import pallas as pl
from jax.experimental.pallas import tpu as pltpu
from jax.experimental.pallas import tpu_sc as plsc

NC = 2
NS = 16
LANES = 16
CHUNK = 128
NW = NC * NS
PW = 128
BN_EPS = 1e-5


def _round_up(x, m):
    return (x + m - 1) // m * m


def _make_seg_sum(n_entries, n_acc, cdup):
    cpt = n_entries // (NW * CHUNK)
    chc = CHUNK // cdup
    rpt = n_acc // NS
    nz = (rpt + CHUNK - 1) // CHUNK
    mesh = plsc.VectorSubcoreMesh(core_axis_name="c", subcore_axis_name="s")

    def body(h_hbm, src_hbm, dst_hbm, cdst_hbm, sums_hbm, cnts_hbm,
             accum, cntacc, src_v, dst_all, cdst_all, dst_cur, cdst_cur,
             rows_a, rows_b, ones_v, gsem_a, gsem_b):
        cid = lax.axis_index("c")
        sid = lax.axis_index("s")
        wid = cid * NS + sid

        zs = jnp.zeros((LANES,), jnp.float32)
        os_ = jnp.ones((LANES,), jnp.float32)

        def fill_z(i, _):
            for k in range(PW // LANES):
                rows_a[i, pl.ds(k * LANES, LANES)] = zs
            return _

        def fill_o(i, _):
            for k in range(PW // LANES):
                ones_v[i, pl.ds(k * LANES, LANES)] = os_
            return _

        lax.fori_loop(0, CHUNK, fill_z, None)
        lax.fori_loop(0, chc, fill_o, None)

        pltpu.sync_copy(src_hbm.at[pl.ds(wid * cpt * CHUNK, cpt * CHUNK)], src_v)
        pltpu.sync_copy(dst_hbm.at[pl.ds(wid * cpt * CHUNK, cpt * CHUNK)], dst_all)
        if cdup != 1:
            pltpu.sync_copy(cdst_hbm.at[pl.ds(wid * cpt * chc, cpt * chc)],
                            cdst_all)

        for z in range(nz):
            r = min(CHUNK, rpt - z * CHUNK)
            r0 = sid * rpt + z * CHUNK
            pltpu.sync_copy(rows_a.at[pl.ds(0, r)], accum.at[pl.ds(r0, r)])
            pltpu.sync_copy(rows_a.at[pl.ds(0, r)], cntacc.at[pl.ds(r0, r)])
        plsc.subcore_barrier()

        def gref(j):
            return h_hbm.at[src_v.at[pl.ds(j * CHUNK, CHUNK)]]

        def start_gather(j, buf, sem):
            pltpu.async_copy(gref(j), buf, sem)

        def wait_gather(j, buf, sem):
            pltpu.make_async_copy(gref(j), buf, sem).wait()

        def scatter_chunk(j, buf):
            csrc = dst_all if cdup == 1 else cdst_all
            for k in range(CHUNK // LANES):
                dst_cur[pl.ds(k * LANES, LANES)] = (
                    dst_all[pl.ds(j * CHUNK + k * LANES, LANES)])
            for k in range(chc // LANES):
                cdst_cur[pl.ds(k * LANES, LANES)] = (
                    csrc[pl.ds(j * chc + k * LANES, LANES)])
            pltpu.sync_copy(buf, accum.at[dst_cur], add=True)
            pltpu.sync_copy(ones_v, cntacc.at[cdst_cur], add=True)

        start_gather(0, rows_a, gsem_a)

        def pair(k, _):
            j0 = 2 * k
            start_gather(j0 + 1, rows_b, gsem_b)
            wait_gather(j0, rows_a, gsem_a)
            scatter_chunk(j0, rows_a)
            start_gather(j0 + 2, rows_a, gsem_a)
            wait_gather(j0 + 1, rows_b, gsem_b)
            scatter_chunk(j0 + 1, rows_b)
            return _

        lax.fori_loop(0, cpt // 2 - 1, pair, None)
        jt = cpt - 2
        start_gather(jt + 1, rows_b, gsem_b)
        wait_gather(jt, rows_a, gsem_a)
        scatter_chunk(jt, rows_a)
        wait_gather(jt + 1, rows_b, gsem_b)
        scatter_chunk(jt + 1, rows_b)
        plsc.subcore_barrier()

        for z in range(nz):
            r = min(CHUNK, rpt - z * CHUNK)
            r0 = sid * rpt + z * CHUNK
            pltpu.sync_copy(accum.at[pl.ds(r0, r)], rows_a.at[pl.ds(0, r)])
            pltpu.sync_copy(rows_a.at[pl.ds(0, r)], sums_hbm.at[cid, pl.ds(r0, r)])
            pltpu.sync_copy(cntacc.at[pl.ds(r0, r)], rows_a.at[pl.ds(0, r)])
            pltpu.sync_copy(rows_a.at[pl.ds(0, r)], cnts_hbm.at[cid, pl.ds(r0, r)])

    return pl.kernel(
        body,
        out_type=(
            jax.ShapeDtypeStruct((NC, n_acc, PW), jnp.float32),
            jax.ShapeDtypeStruct((NC, n_acc, PW), jnp.float32),
        ),
        mesh=mesh,
        scratch_types=[
            pltpu.VMEM_SHARED((n_acc, PW), jnp.float32),
            pltpu.VMEM_SHARED((n_acc, PW), jnp.float32),
            pltpu.VMEM((cpt * CHUNK,), jnp.int32),
            pltpu.VMEM((cpt * CHUNK,), jnp.int32),
            pltpu.VMEM((cpt * chc if cdup != 1 else LANES,), jnp.int32),
            pltpu.VMEM((CHUNK,), jnp.int32),
            pltpu.VMEM((chc,), jnp.int32),
            pltpu.VMEM((CHUNK, PW), jnp.float32),
            pltpu.VMEM((CHUNK, PW), jnp.float32),
            pltpu.VMEM((chc, PW), jnp.float32),
            pltpu.SemaphoreType.DMA,
            pltpu.SemaphoreType.DMA,
        ],
    )


_BN_SCALE = np.float32(np.sqrt(1.0 + BN_EPS))


def _dot(a, b):
    return jax.lax.dot(a, b, preferred_element_type=jnp.float32)


def _mean_from_partials(s_ref, c_ref, n, dup, n_acc):
    s = s_ref[0] + s_ref[1]
    cnt = c_ref[0, :, 0:1] + c_ref[1, :, 0:1]
    if dup == 2:
        s = jnp.concatenate([s[0:n], s[n_acc:n_acc + n]], axis=1)
    else:
        s = s[0:n]
    return s / jnp.maximum(cnt[0:n], 1.0)


def _tc0_body(s_ref, c_ref, x_ref, wl_ref, wr_ref, b_ref, g_ref, be_ref,
              o_ref):
    mean = _mean_from_partials(s_ref, c_ref, 4000, 1, 4096)
    xd = x_ref[0:4000]
    h = _dot(mean, wl_ref[...]) + _dot(xd, wr_ref[...]) + b_ref[...]
    h = (h / _BN_SCALE) * g_ref[...] + be_ref[...]
    h = jnp.where(h > 0, h, 0.2 * h)
    o_ref[0:4000] = h[:, 0:PW]
    o_ref[4000:8000] = h[:, PW:2 * PW]


def _tc1_body(s_ref, c_ref, h_ref, wl_ref, wr_ref, b_ref, g_ref, be_ref,
              w2l_ref, z_ref, hd_ref):
    mean = _mean_from_partials(s_ref, c_ref, 1500, 2, 1536)
    xd = jnp.concatenate([h_ref[0:1500], h_ref[4000:5500]], axis=1)
    h = _dot(mean, wl_ref[...]) + _dot(xd, wr_ref[...]) + b_ref[...]
    h = (h / _BN_SCALE) * g_ref[...] + be_ref[...]
    h = jnp.where(h > 0, h, 0.2 * h)
    z_ref[...] = _dot(h, w2l_ref[...])
    hd_ref[...] = h[0:512]


def _tc2_body(s_ref, c_ref, hd_ref, wr_ref, b_ref, g_ref, be_ref,
              w1_ref, b1_ref, w2_ref, b2_ref, o_ref):
    mean_t = _mean_from_partials(s_ref, c_ref, 512, 1, 640)
    h = mean_t + _dot(hd_ref[...], wr_ref[...]) + b_ref[...]
    h = (h / _BN_SCALE) * g_ref[...] + be_ref[...]
    h = _dot(h, w1_ref[...]) + b1_ref[...]
    h = jnp.where(h > 0, h, 0.2 * h)
    o_ref[...] = _dot(h, w2_ref[...]) + b2_ref[...]


def _prep_edges(ei, n_dst, dup, src_n, dst_n, group, spread):
    e = ei.shape[1]
    e_pad = _round_up(e, group)
    src = ei[0].astype(jnp.int32)
    dst = ei[1].astype(jnp.int32)
    if e_pad != e:
        pad_iota = jnp.arange(e_pad - e, dtype=jnp.int32)
        src = jnp.concatenate([src, pad_iota % 1024])
        dst = jnp.concatenate([dst, n_dst + pad_iota % spread])
    cdst = dst
    if dup:
        src = (src[:, None] + jnp.array([0, src_n], jnp.int32)[None, :]).reshape(-1)
        dst = (dst[:, None] + jnp.array([0, dst_n], jnp.int32)[None, :]).reshape(-1)
    return src, dst, cdst


_SEG0 = _make_seg_sum(131072, 4096, 1)
_SEG1 = _make_seg_sum(98304, 3072, 2)
_SEG2 = _make_seg_sum(16384, 640, 1)


def kernel(x, edge_index_0, edge_index_1, edge_index_2, W0_l, b0, W0_r, g0,
           beta0, W1_l, b1, W1_r, g1, beta1, W2_l, b2, W2_r, g2, beta2,
           W_lin1, b_lin1, W_lin2, b_lin2):
    f32 = jnp.float32

    src0, dst0, cdst0 = _prep_edges(edge_index_0, 4000, False, 0, 0,
                                    NW * CHUNK, 64)
    sums0, cnts0 = _SEG0(x, src0, dst0, cdst0)
    h1p = pl.pallas_call(
        _tc0_body,
        out_shape=jax.ShapeDtypeStruct((8000, PW), f32),
    )(sums0, cnts0, x, W0_l, W0_r, b0, g0, beta0)

    src1, dst1, cdst1 = _prep_edges(edge_index_1, 1500, True, 4000, 1536,
                                    NW * CHUNK // 2, 32)
    sums1, cnts1 = _SEG1(h1p, src1, dst1, cdst1)
    z2, hd = pl.pallas_call(
        _tc1_body,
        out_shape=(
            jax.ShapeDtypeStruct((1500, PW), f32),
            jax.ShapeDtypeStruct((512, 2 * PW), f32),
        ),
    )(sums1, cnts1, h1p, W1_l, W1_r, b1, g1, beta1, W2_l)

    src2, dst2, cdst2 = _prep_edges(edge_index_2, 512, False, 0, 0,
                                    NW * CHUNK, 64)
    sums2, cnts2 = _SEG2(z2, src2, dst2, cdst2)
    out = pl.pallas_call(
        _tc2_body,
        out_shape=jax.ShapeDtypeStruct((512, 1), f32),
    )(sums2, cnts2, hd, W2_r, b2, g2, beta2, W_lin1, b_lin1, W_lin2, b_lin2)
    return out
